# baseline (speedup 1.0000x reference)
import functools

import jax
import jax.numpy as jnp
from jax import lax
from jax.experimental import pallas as pl
from jax.experimental.pallas import tpu as pltpu
from jax.experimental.pallas import tpu_sc as plsc

_R = 20000
_C = 91
_N = _R * _C
_NC = 2
_NS = 16
_NW = _NC * _NS
_L = 16
_U = 8
_CHUNK = 57344
_STRIDE = 56864
_LAST = _N - _CHUNK


def _sc_scan_body(x_hbm, vals_out, idxs_out, buf, val_s, idx_s):
    cid = lax.axis_index("c")
    sid = lax.axis_index("s")
    wid = sid * _NC + cid
    base = jnp.minimum(wid * _STRIDE, _LAST)
    pltpu.sync_copy(x_hbm.at[pl.ds(base, _CHUNK)], buf)

    lanes = lax.iota(jnp.int32, 16)

    def step(i, carry):
        bvs, bis, curs = carry
        new_bvs, new_bis, new_curs = [], [], []
        for j in range(_U):
            v = buf[pl.ds((i * _U + j) * _L, _L)]
            take = v > bvs[j]
            new_bvs.append(jnp.maximum(bvs[j], v))
            new_bis.append(jnp.where(take, curs[j], bis[j]))
            new_curs.append(curs[j] + _U * _L)
        return tuple(new_bvs), tuple(new_bis), tuple(new_curs)

    init = (
        tuple(jnp.full((_L,), -1e30, jnp.float32) for _ in range(_U)),
        tuple(jnp.zeros((_L,), jnp.int32) for _ in range(_U)),
        tuple(base + j * _L + lanes for j in range(_U)),
    )
    bvs, bis, _ = lax.fori_loop(0, _CHUNK // (_U * _L), step, init)

    bvs, bis = list(bvs), list(bis)
    while len(bvs) > 1:
        nv, ni = [], []
        for k in range(0, len(bvs), 2):
            va, vb = bvs[k], bvs[k + 1]
            ia, ib = bis[k], bis[k + 1]
            take_b = (vb > va) | ((vb == va) & (ib < ia))
            nv.append(jnp.where(take_b, vb, va))
            ni.append(jnp.where(take_b, ib, ia))
        bvs, bis = nv, ni

    val_s[...] = bvs[0]
    idx_s[...] = bis[0]
    pltpu.sync_copy(val_s, vals_out.at[pl.ds(wid * _L, _L)])
    pltpu.sync_copy(idx_s, idxs_out.at[pl.ds(wid * _L, _L)])


@functools.lru_cache(maxsize=None)
def _build_sc_scan():
    return pl.kernel(
        _sc_scan_body,
        out_type=(
            jax.ShapeDtypeStruct((_NW * _L,), jnp.float32),
            jax.ShapeDtypeStruct((_NW * _L,), jnp.int32),
        ),
        mesh=plsc.VectorSubcoreMesh(
            core_axis_name="c", subcore_axis_name="s",
            num_cores=_NC, num_subcores=_NS,
        ),
        scratch_types=(
            pltpu.VMEM((_CHUNK,), jnp.float32),
            pltpu.VMEM((_L,), jnp.float32),
            pltpu.VMEM((_L,), jnp.int32),
        ),
    )


def _tc_finish_body(v_ref, i_ref, score_ref, label_ref):
    v = v_ref[...]
    idx = i_ref[...]
    m = jnp.max(v)
    sel = jnp.where(v == m, idx, jnp.int32(2**31 - 1))
    mi = jnp.min(sel, keepdims=True).reshape(1, 1)
    score_ref[...] = 1.0 / (1.0 + jnp.exp(-jnp.max(v, keepdims=True).reshape(1, 1)))
    label_ref[...] = mi % _C


def _tc_finish(vals, idxs):
    return pl.pallas_call(
        _tc_finish_body,
        out_shape=(
            jax.ShapeDtypeStruct((1, 1), jnp.float32),
            jax.ShapeDtypeStruct((1, 1), jnp.int32),
        ),
    )(vals, idxs)


def kernel(pred_logits, pred_boxes):
    del pred_boxes
    xf = pred_logits.reshape(-1)
    vals, idxs = _build_sc_scan()(xf)
    score, label = _tc_finish(vals.reshape(4, 128), idxs.reshape(4, 128))
    return (score.reshape(1), label.reshape(1))


# speedup vs baseline: 8.2453x; 8.2453x over previous
import functools

import jax
import jax.numpy as jnp
from jax import lax
from jax.experimental import pallas as pl
from jax.experimental.pallas import tpu as pltpu
from jax.experimental.pallas import tpu_sc as plsc

_R = 20000
_C = 91
_N = _R * _C
_NC = 2
_NS = 16
_NW = _NC * _NS
_L = 16
_U = 8
_CHUNK = 57344
_STRIDE = 56864
_LAST = _N - _CHUNK


def _sc_scan_body(x_hbm, vals_out, idxs_out, buf, val_s, idx_s):
    cid = lax.axis_index("c")
    sid = lax.axis_index("s")
    wid = sid * _NC + cid
    base = jnp.minimum(wid * _STRIDE, _LAST)
    pltpu.sync_copy(x_hbm.at[pl.ds(base, _CHUNK)], buf)

    lanes = lax.iota(jnp.int32, 16)

    def step(i, carry):
        bvs, bis, curs = carry
        new_bvs, new_bis, new_curs = [], [], []
        for j in range(_U):
            v = buf[pl.ds((i * _U + j) * _L, _L)]
            take = v > bvs[j]
            new_bvs.append(jnp.maximum(bvs[j], v))
            new_bis.append(jnp.where(take, curs[j], bis[j]))
            new_curs.append(curs[j] + _U * _L)
        return tuple(new_bvs), tuple(new_bis), tuple(new_curs)

    init = (
        tuple(jnp.full((_L,), -1e30, jnp.float32) for _ in range(_U)),
        tuple(jnp.zeros((_L,), jnp.int32) for _ in range(_U)),
        tuple(base + j * _L + lanes for j in range(_U)),
    )
    bvs, bis, _ = lax.fori_loop(0, _CHUNK // (_U * _L), step, init)

    bvs, bis = list(bvs), list(bis)
    while len(bvs) > 1:
        nv, ni = [], []
        for k in range(0, len(bvs), 2):
            va, vb = bvs[k], bvs[k + 1]
            ia, ib = bis[k], bis[k + 1]
            take_b = (vb > va) | ((vb == va) & (ib < ia))
            nv.append(jnp.where(take_b, vb, va))
            ni.append(jnp.where(take_b, ib, ia))
        bvs, bis = nv, ni

    val_s[...] = bvs[0]
    idx_s[...] = bis[0]
    pltpu.sync_copy(val_s, vals_out.at[pl.ds(wid * _L, _L)])
    pltpu.sync_copy(idx_s, idxs_out.at[pl.ds(wid * _L, _L)])


@functools.lru_cache(maxsize=None)
def _build_sc_scan():
    return pl.kernel(
        _sc_scan_body,
        out_type=(
            jax.ShapeDtypeStruct((_NW * _L,), jnp.float32),
            jax.ShapeDtypeStruct((_NW * _L,), jnp.int32),
        ),
        mesh=plsc.VectorSubcoreMesh(
            core_axis_name="c", subcore_axis_name="s",
            num_cores=_NC, num_subcores=_NS,
        ),
        scratch_types=(
            pltpu.VMEM((_CHUNK,), jnp.float32),
            pltpu.VMEM((_L,), jnp.float32),
            pltpu.VMEM((_L,), jnp.int32),
        ),
    )


def _tc_finish_body(v_ref, i_ref, score_ref, label_ref):
    v = v_ref[...]
    idx = i_ref[...]
    m = jnp.max(v)
    sel = jnp.where(v == m, idx, jnp.int32(2**31 - 1))
    mi = jnp.min(sel, keepdims=True).reshape(1, 1)
    score_ref[...] = 1.0 / (1.0 + jnp.exp(-jnp.max(v, keepdims=True).reshape(1, 1)))
    label_ref[...] = mi % _C


def _tc_finish(vals, idxs):
    return pl.pallas_call(
        _tc_finish_body,
        out_shape=(
            jax.ShapeDtypeStruct((1, 1), jnp.float32),
            jax.ShapeDtypeStruct((1, 1), jnp.int32),
        ),
    )(vals, idxs)


def kernel(pred_logits, pred_boxes):
    del pred_boxes
    xf = pred_logits[0].reshape(-1)
    vals, idxs = _build_sc_scan()(xf)
    score, label = _tc_finish(vals.reshape(4, 128), idxs.reshape(4, 128))
    return (score.reshape(1), label.reshape(1))
